# Initial kernel scaffold; baseline (speedup 1.0000x reference)
import jax
import jax.numpy as jnp
from jax.experimental import pallas as pl


def kernel(edge_index_drop, edge_index, features, preference):
    raise NotImplementedError("write your pallas kernel here")


# speedup vs baseline: 7.9385x; 7.9385x over previous
import functools

import jax
import jax.numpy as jnp
from jax import lax
from jax.experimental import pallas as pl
from jax.experimental.pallas import tpu as pltpu
from jax.experimental.pallas import tpu_sc as plsc

N_NODES = 10000
D_FEAT = 128
HALF = 64
N_EDGES = 320000

N_PAD = 10240
RPT = N_PAD // 16
CH = 128
NCH = 158
E_TILE = NCH * CH
E_PAD = 16 * E_TILE


def _normalize_body(f_ref, o_ref):
    x = f_ref[...]
    n2 = jnp.sum(x * x, axis=1, keepdims=True)
    nrm = jnp.sqrt(n2)
    o_ref[...] = x / jnp.maximum(nrm, 1e-12)


def _tc_normalize(features):
    return pl.pallas_call(
        _normalize_body,
        grid=(10,),
        in_specs=[pl.BlockSpec((1000, 128), lambda i: (i, 0))],
        out_specs=pl.BlockSpec((1000, 128), lambda i: (i, 0)),
        out_shape=jax.ShapeDtypeStruct((N_NODES, D_FEAT), jnp.float32),
    )(features)


def _rsqrt16(v):
    i = lax.bitcast_convert_type(v, jnp.int32)
    y = lax.bitcast_convert_type(jnp.int32(0x5F3759DF) - (i >> 1), jnp.float32)
    for _ in range(3):
        y = y * (1.5 - 0.5 * v * y * y)
    return y


def _sc_body(row_hbm, col_hbm, x_hbm, out_hbm, h_hbm,
             row_v, col_v, norm_v, dis_v, idx_v, g0, g1, zb_v,
             a_sh, deg_sh, sem):
    cid = lax.axis_index("c")
    sid = lax.axis_index("s")
    base_r = sid * RPT

    z16 = jnp.zeros((16,), jnp.float32)
    one16 = jnp.ones((16,), jnp.float32)

    def zrow(i, _):
        for k in range(4):
            g0[i, pl.ds(16 * k, 16)] = z16
        return 0
    lax.fori_loop(0, CH, zrow, 0)

    def zzb(i, _):
        zb_v[pl.ds(i * 16, 16)] = z16
        return 0
    lax.fori_loop(0, RPT // 16, zzb, 0)

    for s in range(RPT // CH):
        pltpu.sync_copy(g0, a_sh.at[pl.ds(base_r + CH * s, CH)])
    pltpu.sync_copy(zb_v, deg_sh.at[pl.ds(base_r, RPT)])
    plsc.subcore_barrier()

    pltpu.sync_copy(row_hbm.at[sid], row_v)
    pltpu.sync_copy(col_hbm.at[sid], col_v)

    def degchunk(j, _):
        for k in range(8):
            sl = pl.ds(16 * k, 16)
            r = row_v[j, sl]
            c = col_v[j, sl]
            norm_v[j, sl] = jnp.where(r != c, one16, z16)
        pltpu.sync_copy(norm_v.at[j], deg_sh.at[row_v.at[j]], add=True)
        return 0
    lax.fori_loop(0, NCH, degchunk, 0)
    plsc.subcore_barrier()

    pltpu.sync_copy(deg_sh, dis_v)

    def disrow(i, _):
        sl = pl.ds(16 * i, 16)
        dis_v[sl] = _rsqrt16(dis_v[sl])
        return 0
    lax.fori_loop(0, N_PAD // 16, disrow, 0)

    def normchunk(j, _):
        for k in range(8):
            sl = pl.ds(16 * k, 16)
            r = row_v[j, sl]
            c = col_v[j, sl]
            dr = plsc.load_gather(dis_v, [r])
            dc = plsc.load_gather(dis_v, [c])
            norm_v[j, sl] = dr * dc * norm_v[j, sl]
            row_v[j, sl] = r * 2 + cid
        return 0
    lax.fori_loop(0, NCH, normchunk, 0)

    def conv(src_hbm, acc_sh):
        def chunk(j, _):
            pltpu.async_copy(src_hbm.at[row_v.at[j]], g0, sem).wait()

            def scale_g(g, _):
                nv = norm_v[j, pl.ds(16 * g, 16)]
                base = 16 * g
                for t in range(16):
                    s = nv[t]
                    for k in range(4):
                        sl = pl.ds(16 * k, 16)
                        g0[base + t, sl] = g0[base + t, sl] * s
                return 0
            lax.fori_loop(0, CH // 16, scale_g, 0)
            pltpu.sync_copy(g0, acc_sh.at[col_v.at[j]], add=True)
            return 0
        lax.fori_loop(0, NCH, chunk, 0)

    conv(x_hbm, a_sh)
    plsc.subcore_barrier()

    hb = pl.multiple_of(cid * N_PAD + base_r, CH)
    for s in range(RPT // CH):
        pltpu.sync_copy(a_sh.at[pl.ds(base_r + CH * s, CH)],
                        h_hbm.at[pl.ds(hb + CH * s, CH)])

    def zrow2(i, _):
        for k in range(4):
            g0[i, pl.ds(16 * k, 16)] = z16
        return 0
    lax.fori_loop(0, CH, zrow2, 0)
    for s in range(RPT // CH):
        pltpu.sync_copy(g0, a_sh.at[pl.ds(base_r + CH * s, CH)])

    def adjchunk(j, _):
        for k in range(8):
            sl = pl.ds(16 * k, 16)
            r = (row_v[j, sl] - cid) >> 1
            row_v[j, sl] = r + cid * N_PAD
        return 0
    lax.fori_loop(0, NCH, adjchunk, 0)
    plsc.subcore_barrier()

    conv(h_hbm, a_sh)
    plsc.subcore_barrier()

    for s in range(RPT // CH):
        b0 = base_r + CH * s
        for k in range(8):
            lane = lax.iota(jnp.int32, 16) + (b0 + 16 * k)
            idx_v[pl.ds(16 * k, 16)] = lane * 2 + cid
        pltpu.async_copy(x_hbm.at[idx_v], g0, sem).wait()
        pltpu.sync_copy(a_sh.at[pl.ds(b0, CH)], g1)

        def addrow(i, _):
            for k in range(4):
                sl = pl.ds(16 * k, 16)
                g0[i, sl] = g0[i, sl] + g1[i, sl]
            return 0
        lax.fori_loop(0, CH, addrow, 0)
        pltpu.sync_copy(h_hbm.at[pl.ds(hb + CH * s, CH)], g1)
        lax.fori_loop(0, CH, addrow, 0)
        pltpu.sync_copy(g0, out_hbm.at[cid, pl.ds(b0, CH)])


def _sc_gcn(row_t, col_t, xflat):
    mesh = plsc.VectorSubcoreMesh(core_axis_name="c", subcore_axis_name="s")
    return pl.kernel(
        _sc_body,
        out_type=[
            jax.ShapeDtypeStruct((2, N_PAD, HALF), jnp.float32),
            jax.ShapeDtypeStruct((2 * N_PAD, HALF), jnp.float32),
        ],
        mesh=mesh,
        compiler_params=pltpu.CompilerParams(needs_layout_passes=False,
                                             use_tc_tiling_on_sc=False),
        scratch_types=[
            pltpu.VMEM((NCH, CH), jnp.int32),
            pltpu.VMEM((NCH, CH), jnp.int32),
            pltpu.VMEM((NCH, CH), jnp.float32),
            pltpu.VMEM((N_PAD,), jnp.float32),
            pltpu.VMEM((CH,), jnp.int32),
            pltpu.VMEM((CH, HALF), jnp.float32),
            pltpu.VMEM((CH, HALF), jnp.float32),
            pltpu.VMEM((RPT,), jnp.float32),
            pltpu.VMEM_SHARED((N_PAD, HALF), jnp.float32),
            pltpu.VMEM_SHARED((N_PAD,), jnp.float32),
            pltpu.SemaphoreType.DMA,
        ],
    )(row_t, col_t, xflat)


def kernel(edge_index_drop, edge_index, features, preference):
    del edge_index_drop
    x = _tc_normalize(features.astype(jnp.float32))

    xpad = jnp.pad(x, ((0, N_PAD - N_NODES), (0, 0)))
    xflat = xpad.reshape(2 * N_PAD, HALF)

    ei = edge_index.astype(jnp.int32)
    rowp = jnp.pad(ei[0], (0, E_PAD - N_EDGES)).reshape(16, NCH, CH)
    colp = jnp.pad(ei[1], (0, E_PAD - N_EDGES)).reshape(16, NCH, CH)

    out_split, _h = _sc_gcn(rowp, colp, xflat)
    x_hat = out_split.transpose(1, 0, 2).reshape(N_PAD, D_FEAT)[:N_NODES]
    return (x_hat, preference)


# speedup vs baseline: 10.5660x; 1.3310x over previous
import functools

import jax
import jax.numpy as jnp
from jax import lax
from jax.experimental import pallas as pl
from jax.experimental.pallas import tpu as pltpu
from jax.experimental.pallas import tpu_sc as plsc

N_NODES = 10000
D_FEAT = 128
HALF = 64
N_EDGES = 320000

N_PAD = 10240
RPT = N_PAD // 16
CH = 128
NCH = 158
E_TILE = NCH * CH
E_PAD = 16 * E_TILE


def _normalize_body(f_ref, o_ref):
    x = f_ref[...]
    n2 = jnp.sum(x * x, axis=1, keepdims=True)
    nrm = jnp.sqrt(n2)
    o_ref[...] = x / jnp.maximum(nrm, 1e-12)


def _tc_normalize(features):
    return pl.pallas_call(
        _normalize_body,
        grid=(10,),
        in_specs=[pl.BlockSpec((1000, 128), lambda i: (i, 0))],
        out_specs=pl.BlockSpec((1000, 128), lambda i: (i, 0)),
        out_shape=jax.ShapeDtypeStruct((N_NODES, D_FEAT), jnp.float32),
    )(features)


def _rsqrt16(v):
    i = lax.bitcast_convert_type(v, jnp.int32)
    y = lax.bitcast_convert_type(jnp.int32(0x5F3759DF) - (i >> 1), jnp.float32)
    for _ in range(3):
        y = y * (1.5 - 0.5 * v * y * y)
    return y


def _sc_body(row_hbm, col_hbm, x_hbm, out_hbm, h_hbm,
             row_v, col_v, norm_v, dis_v, idx_v, g0, g1, zb_v,
             a_sh, deg_sh, sem, sg0, sg1, ss0, ss1):
    cid = lax.axis_index("c")
    sid = lax.axis_index("s")
    base_r = sid * RPT

    z16 = jnp.zeros((16,), jnp.float32)
    one16 = jnp.ones((16,), jnp.float32)

    def zrow(i, _):
        for k in range(4):
            g0[i, pl.ds(16 * k, 16)] = z16
        return 0
    lax.fori_loop(0, CH, zrow, 0)

    def zzb(i, _):
        zb_v[pl.ds(i * 16, 16)] = z16
        return 0
    lax.fori_loop(0, RPT // 16, zzb, 0)

    icps = [pltpu.async_copy(g0, a_sh.at[pl.ds(base_r + CH * s, CH)], sg0)
            for s in range(RPT // CH)]
    icps.append(pltpu.async_copy(zb_v, deg_sh.at[pl.ds(base_r, RPT)], sg0))
    for c in icps:
        c.wait()
    plsc.subcore_barrier()

    pltpu.sync_copy(row_hbm.at[sid], row_v)
    pltpu.sync_copy(col_hbm.at[sid], col_v)

    def degchunk(j, _):
        for k in range(8):
            sl = pl.ds(16 * k, 16)
            r = row_v[j, sl]
            c = col_v[j, sl]
            norm_v[j, sl] = jnp.where(r != c, one16, z16)

        @pl.when(j >= 4)
        def _():
            pltpu.make_async_copy(norm_v.at[j - 4],
                                  deg_sh.at[row_v.at[j - 4]], sem).wait()
        pltpu.async_copy(norm_v.at[j], deg_sh.at[row_v.at[j]], sem, add=True)
        return 0
    lax.fori_loop(0, NCH, degchunk, 0)
    for j in range(NCH - 4, NCH):
        pltpu.make_async_copy(norm_v.at[j], deg_sh.at[row_v.at[j]], sem).wait()
    plsc.subcore_barrier()

    pltpu.sync_copy(deg_sh, dis_v)

    def disrow(i, _):
        sl = pl.ds(16 * i, 16)
        dis_v[sl] = _rsqrt16(dis_v[sl])
        return 0
    lax.fori_loop(0, N_PAD // 16, disrow, 0)

    def normchunk(j, _):
        for k in range(8):
            sl = pl.ds(16 * k, 16)
            r = row_v[j, sl]
            c = col_v[j, sl]
            dr = plsc.load_gather(dis_v, [r])
            dc = plsc.load_gather(dis_v, [c])
            norm_v[j, sl] = dr * dc * norm_v[j, sl]
            row_v[j, sl] = r * 2 + cid
        return 0
    lax.fori_loop(0, NCH, normchunk, 0)

    def scale(gbuf, j):
        def scale_g(g, _):
            nv = norm_v[j, pl.ds(16 * g, 16)]
            base = 16 * g
            for t in range(16):
                sv = nv[t]
                for k in range(4):
                    sl = pl.ds(16 * k, 16)
                    gbuf[base + t, sl] = gbuf[base + t, sl] * sv
            return 0
        lax.fori_loop(0, CH // 16, scale_g, 0)

    def conv(src_hbm, acc_sh):
        pltpu.async_copy(src_hbm.at[row_v.at[0]], g0, sg0)
        pltpu.async_copy(src_hbm.at[row_v.at[1]], g1, sg1)

        def pair(i, _):
            j0 = 2 * i
            j1 = j0 + 1
            pltpu.make_async_copy(src_hbm.at[row_v.at[j0]], g0, sg0).wait()
            scale(g0, j0)
            s0 = pltpu.async_copy(g0, acc_sh.at[col_v.at[j0]], ss0, add=True)
            pltpu.make_async_copy(src_hbm.at[row_v.at[j1]], g1, sg1).wait()
            scale(g1, j1)
            s1 = pltpu.async_copy(g1, acc_sh.at[col_v.at[j1]], ss1, add=True)
            s0.wait()
            pltpu.async_copy(src_hbm.at[row_v.at[j0 + 2]], g0, sg0)
            s1.wait()
            pltpu.async_copy(src_hbm.at[row_v.at[j1 + 2]], g1, sg1)
            return 0
        lax.fori_loop(0, NCH // 2 - 1, pair, 0)

        j0 = NCH - 2
        j1 = NCH - 1
        pltpu.make_async_copy(src_hbm.at[row_v.at[j0]], g0, sg0).wait()
        scale(g0, j0)
        s0 = pltpu.async_copy(g0, acc_sh.at[col_v.at[j0]], ss0, add=True)
        pltpu.make_async_copy(src_hbm.at[row_v.at[j1]], g1, sg1).wait()
        scale(g1, j1)
        s1 = pltpu.async_copy(g1, acc_sh.at[col_v.at[j1]], ss1, add=True)
        s0.wait()
        s1.wait()

    conv(x_hbm, a_sh)
    plsc.subcore_barrier()

    hb = pl.multiple_of(cid * N_PAD + base_r, CH)
    hcps = [pltpu.async_copy(a_sh.at[pl.ds(base_r + CH * s, CH)],
                             h_hbm.at[pl.ds(hb + CH * s, CH)], sg0)
            for s in range(RPT // CH)]
    for c in hcps:
        c.wait()

    def zrow2(i, _):
        for k in range(4):
            g0[i, pl.ds(16 * k, 16)] = z16
        return 0
    lax.fori_loop(0, CH, zrow2, 0)
    zcps = [pltpu.async_copy(g0, a_sh.at[pl.ds(base_r + CH * s, CH)], sg1)
            for s in range(RPT // CH)]
    for c in zcps:
        c.wait()

    def adjchunk(j, _):
        for k in range(8):
            sl = pl.ds(16 * k, 16)
            r = (row_v[j, sl] - cid) >> 1
            row_v[j, sl] = r + cid * N_PAD
        return 0
    lax.fori_loop(0, NCH, adjchunk, 0)
    plsc.subcore_barrier()

    conv(h_hbm, a_sh)
    plsc.subcore_barrier()

    for s in range(RPT // CH):
        b0 = base_r + CH * s
        for k in range(8):
            lane = lax.iota(jnp.int32, 16) + (b0 + 16 * k)
            idx_v[pl.ds(16 * k, 16)] = lane * 2 + cid
        pltpu.async_copy(x_hbm.at[idx_v], g0, sem).wait()
        pltpu.sync_copy(a_sh.at[pl.ds(b0, CH)], g1)

        def addrow(i, _):
            for k in range(4):
                sl = pl.ds(16 * k, 16)
                g0[i, sl] = g0[i, sl] + g1[i, sl]
            return 0
        lax.fori_loop(0, CH, addrow, 0)
        pltpu.sync_copy(h_hbm.at[pl.ds(hb + CH * s, CH)], g1)
        lax.fori_loop(0, CH, addrow, 0)
        pltpu.sync_copy(g0, out_hbm.at[cid, pl.ds(b0, CH)])


def _sc_gcn(row_t, col_t, xflat):
    mesh = plsc.VectorSubcoreMesh(core_axis_name="c", subcore_axis_name="s")
    return pl.kernel(
        _sc_body,
        out_type=[
            jax.ShapeDtypeStruct((2, N_PAD, HALF), jnp.float32),
            jax.ShapeDtypeStruct((2 * N_PAD, HALF), jnp.float32),
        ],
        mesh=mesh,
        compiler_params=pltpu.CompilerParams(needs_layout_passes=False,
                                             use_tc_tiling_on_sc=False),
        scratch_types=[
            pltpu.VMEM((NCH, CH), jnp.int32),
            pltpu.VMEM((NCH, CH), jnp.int32),
            pltpu.VMEM((NCH, CH), jnp.float32),
            pltpu.VMEM((N_PAD,), jnp.float32),
            pltpu.VMEM((CH,), jnp.int32),
            pltpu.VMEM((CH, HALF), jnp.float32),
            pltpu.VMEM((CH, HALF), jnp.float32),
            pltpu.VMEM((RPT,), jnp.float32),
            pltpu.VMEM_SHARED((N_PAD, HALF), jnp.float32),
            pltpu.VMEM_SHARED((N_PAD,), jnp.float32),
            pltpu.SemaphoreType.DMA,
            pltpu.SemaphoreType.DMA,
            pltpu.SemaphoreType.DMA,
            pltpu.SemaphoreType.DMA,
            pltpu.SemaphoreType.DMA,
        ],
    )(row_t, col_t, xflat)


def kernel(edge_index_drop, edge_index, features, preference):
    del edge_index_drop
    x = _tc_normalize(features.astype(jnp.float32))

    xpad = jnp.pad(x, ((0, N_PAD - N_NODES), (0, 0)))
    xflat = xpad.reshape(2 * N_PAD, HALF)

    ei = edge_index.astype(jnp.int32)
    rowp = jnp.pad(ei[0], (0, E_PAD - N_EDGES)).reshape(16, NCH, CH)
    colp = jnp.pad(ei[1], (0, E_PAD - N_EDGES)).reshape(16, NCH, CH)

    out_split, _h = _sc_gcn(rowp, colp, xflat)
    x_hat = out_split.transpose(1, 0, 2).reshape(N_PAD, D_FEAT)[:N_NODES]
    return (x_hat, preference)
